# Initial kernel scaffold; baseline (speedup 1.0000x reference)
import jax
import jax.numpy as jnp
from jax.experimental import pallas as pl


def kernel(x, edge_index, W_gat, att_src, att_dst, bias_gat, w_ih, w_hh, b_ih, b_hh, ln_gamma, ln_beta):
    raise NotImplementedError("write your pallas kernel here")


# speedup vs baseline: 1.0548x; 1.0548x over previous
import functools

import jax
import jax.numpy as jnp
from jax.experimental import pallas as pl
from jax.experimental.pallas import tpu as pltpu

B, T, N, F = 2, 8, 5000, 128
H = 128
HEADS = 4
DH = H // HEADS
E = 500000
NF = B * T * N


def _ln_kernel(ctx_ref, gamma_ref, beta_ref, out_ref):
    ctx = ctx_ref[...]
    mean = jnp.mean(ctx, axis=-1, keepdims=True)
    var = jnp.mean((ctx - mean) ** 2, axis=-1, keepdims=True)
    y = (ctx - mean) / jnp.sqrt(var + 1e-5) * gamma_ref[...] + beta_ref[...]
    out_ref[...] = jnp.broadcast_to(y[:, None, :], (B, N, H))


def kernel(x, edge_index, W_gat, att_src, att_dst, bias_gat, w_ih, w_hh, b_ih, b_hh, ln_gamma, ln_beta):
    x_flat = x.reshape(-1, F)
    n = x_flat.shape[0]
    h = (x_flat @ W_gat).reshape(n, HEADS, DH)
    src = edge_index[0]
    dst = edge_index[1]
    loop = jnp.arange(n, dtype=src.dtype)
    src = jnp.concatenate([src, loop])
    dst = jnp.concatenate([dst, loop])
    a_src = (h * att_src[None, :, :]).sum(-1)
    a_dst = (h * att_dst[None, :, :]).sum(-1)
    e = a_src[src] + a_dst[dst]
    e = jnp.where(e > 0, e, 0.2 * e)
    w = jnp.exp(e)
    ssum = jax.ops.segment_sum(w, dst, num_segments=n)
    msg = h[src] * w[:, :, None]
    out = jax.ops.segment_sum(msg, dst, num_segments=n)
    out = out / (ssum[:, :, None] + 1e-16)
    hgat = out.reshape(n, HEADS * DH) + bias_gat
    hgat = jax.nn.relu(hgat)
    hseq = hgat.reshape(B, T, N, -1).mean(axis=2)

    h0 = jnp.zeros((B, H), dtype=x.dtype)
    c0 = jnp.zeros((B, H), dtype=x.dtype)

    def step(carry, xt):
        hprev, cprev = carry
        g = xt @ w_ih.T + hprev @ w_hh.T + b_ih + b_hh
        i, f, gg, o = jnp.split(g, 4, axis=-1)
        i = jax.nn.sigmoid(i)
        f = jax.nn.sigmoid(f)
        gg = jnp.tanh(gg)
        o = jax.nn.sigmoid(o)
        c = f * cprev + i * gg
        hnew = o * jnp.tanh(c)
        return (hnew, c), None

    (context, _), _ = jax.lax.scan(step, (h0, c0), jnp.swapaxes(hseq, 0, 1))

    out = pl.pallas_call(
        _ln_kernel,
        out_shape=jax.ShapeDtypeStruct((B, N, H), jnp.float32),
    )(context, ln_gamma, ln_beta)
    return out


# speedup vs baseline: 21.7799x; 20.6488x over previous
import functools

import jax
import jax.numpy as jnp
from jax import lax
from jax.experimental import pallas as pl
from jax.experimental.pallas import tpu as pltpu
from jax.experimental.pallas import tpu_sc as plsc

B, T, N, F = 2, 8, 5000, 128
H = 128
HEADS = 4
DH = H // HEADS
E = 500000
NF = B * T * N

NTILES = 32
BW = 128
NBATCH = 128
EPT = NBATCH * BW
E_PAD = NTILES * EPT
STRIPE = NF // 16
ZCH = 250

RB1 = 2000
RB3 = 1000


def _proj_body(x_ref, w_ref, a_ref, s0, s1, s2, s3, ssrc_ref, sdst_ref):
    xb = x_ref[...]
    hb = jnp.dot(xb, w_ref[...], preferred_element_type=jnp.float32)
    aux = jnp.dot(hb, a_ref[...], preferred_element_type=jnp.float32)
    es = aux[:, 0:4] + aux[:, 4:8]
    wself = jnp.exp(jnp.where(es > 0, es, 0.2 * es))
    pad8 = jnp.zeros((xb.shape[0], 8), jnp.float32)
    ssrc_ref[...] = jnp.concatenate([aux[:, 0:4], wself, pad8], axis=1)
    sdst_ref[...] = jnp.concatenate([aux[:, 4:8], wself, pad8], axis=1)
    for h, r in enumerate((s0, s1, s2, s3)):
        r[...] = hb[:, DH * h:DH * h + DH].astype(jnp.bfloat16)


def _project(x_flat, W_gat, Aall):
    grid = NF // RB1
    return pl.pallas_call(
        _proj_body,
        grid=(grid,),
        in_specs=[
            pl.BlockSpec((RB1, F), lambda i: (i, 0)),
            pl.BlockSpec((F, H), lambda i: (0, 0)),
            pl.BlockSpec((F, 8), lambda i: (0, 0)),
        ],
        out_specs=[pl.BlockSpec((RB1, DH), lambda i: (i, 0))] * 4
        + [pl.BlockSpec((RB1, 16), lambda i: (i, 0))] * 2,
        out_shape=[jax.ShapeDtypeStruct((NF, DH), jnp.bfloat16)] * 4
        + [jax.ShapeDtypeStruct((NF, 16), jnp.float32)] * 2,
    )(x_flat, W_gat, Aall)


def _edge_body(src_hbm, dst_hbm, ssrc_hbm, sdst_hbm, t0, t1, t2, t3,
               m0, m1, m2, m3, ss, w_hbm,
               sbi, dbi, wbuf, sbuf, dbuf, rbuf, zbuf, acc, sem):
    sc = lax.axis_index("c")
    sid = lax.axis_index("s")
    wid = sc * 16 + sid
    row0 = sid * STRIPE

    lane = lax.iota(jnp.int32, 16)
    off4 = jnp.minimum(lane, 3)
    qmask = lane < 4

    cg = pltpu.sync_copy

    for i in range(ZCH):
        zbuf[i] = jnp.zeros((DH,), jnp.bfloat16)

    def _zero_stripe():
        for k in range(STRIPE // ZCH):
            cg(zbuf, acc.at[pl.ds(row0 + k * ZCH, ZCH)])

    _zero_stripe()
    plsc.subcore_barrier()

    ebase = wid * EPT

    def _batch_a(b, _):
        row = wid * NBATCH + b
        cg(src_hbm.at[pl.ds(row, 1)], sbi)
        cg(dst_hbm.at[pl.ds(row, 1)], dbi)
        cg(ssrc_hbm.at[sbi.at[0]], sbuf)
        cg(sdst_hbm.at[dbi.at[0]], dbuf)

        def _edge(j, _):
            sv = sbuf[j]
            dv = dbuf[j]
            e = sv + dv
            e = jnp.where(e > 0, e, 0.2 * e)
            w = jnp.exp(e)
            w = jnp.where(ebase + b * BW + j < E, w, 0.0)
            idx = jnp.where(qmask, 4 * j + lane, 4 * BW + 8)
            plsc.store_scatter(wbuf, [idx], w)
            return 0

        lax.fori_loop(0, BW, _edge, 0, unroll=8)
        cg(wbuf.at[pl.ds(0, 4 * BW)], w_hbm.at[pl.ds(row * 4 * BW, 4 * BW)])
        return 0

    lax.fori_loop(0, NBATCH, _batch_a, 0)

    for h, (tab, mout) in enumerate(((t0, m0), (t1, m1), (t2, m2), (t3, m3))):
        def _batch_h(b, _, h=h, tab=tab):
            row = wid * NBATCH + b
            cg(src_hbm.at[pl.ds(row, 1)], sbi)
            cg(dst_hbm.at[pl.ds(row, 1)], dbi)
            cg(w_hbm.at[pl.ds(row * 4 * BW, 4 * BW)], wbuf.at[pl.ds(0, 4 * BW)])
            cg(tab.at[sbi.at[0]], rbuf)

            def _edge(j, _):
                wv = plsc.load_gather(wbuf, [jnp.full((16,), 4 * j + h,
                                                      jnp.int32)])
                wb = plsc.pack(wv, wv, format=plsc.PackFormat.INTERLEAVED)
                rbuf[j] = rbuf[j] * wb
                return 0

            lax.fori_loop(0, BW, _edge, 0, unroll=16)
            cg(rbuf, acc.at[dbi.at[0]], add=True)
            return 0

        lax.fori_loop(0, NBATCH, _batch_h, 0)
        plsc.subcore_barrier()
        for k in range(STRIPE // ZCH):
            cg(acc.at[pl.ds(row0 + k * ZCH, ZCH)],
               mout.at[pl.ds(sc * NF + row0 + k * ZCH, ZCH)])
        _zero_stripe()
        plsc.subcore_barrier()

    def _batch_s(b, _):
        row = wid * NBATCH + b
        cg(dst_hbm.at[pl.ds(row, 1)], dbi)
        cg(w_hbm.at[pl.ds(row * 4 * BW, 4 * BW)], wbuf.at[pl.ds(0, 4 * BW)])

        def _edge(j, _):
            wv = plsc.load_gather(wbuf, [4 * j + off4])
            wb = plsc.pack(wv, wv, format=plsc.PackFormat.INTERLEAVED)
            rbuf[j] = wb
            return 0

        lax.fori_loop(0, BW, _edge, 0, unroll=16)
        cg(rbuf, acc.at[dbi.at[0]], add=True)
        return 0

    lax.fori_loop(0, NBATCH, _batch_s, 0)
    plsc.subcore_barrier()
    for k in range(STRIPE // ZCH):
        cg(acc.at[pl.ds(row0 + k * ZCH, ZCH)],
           ss.at[pl.ds(sc * NF + row0 + k * ZCH, ZCH)])


def _edge_phase(srcp, dstp, ssrc, sdst, t0, t1, t2, t3):
    mesh = plsc.VectorSubcoreMesh(core_axis_name="c", subcore_axis_name="s")
    run = pl.kernel(
        _edge_body,
        out_type=[jax.ShapeDtypeStruct((2 * NF, DH), jnp.bfloat16)] * 5
        + [jax.ShapeDtypeStruct((NTILES * NBATCH * 4 * BW,), jnp.float32)],
        mesh=mesh,
        scratch_types=[
            pltpu.VMEM((1, BW), jnp.int32),
            pltpu.VMEM((1, BW), jnp.int32),
            pltpu.VMEM((4 * BW + 32,), jnp.float32),
            pltpu.VMEM((BW, 16), jnp.float32),
            pltpu.VMEM((BW, 16), jnp.float32),
            pltpu.VMEM((BW, DH), jnp.bfloat16),
            pltpu.VMEM((ZCH, DH), jnp.bfloat16),
            pltpu.VMEM_SHARED((NF, DH), jnp.bfloat16),
            pltpu.SemaphoreType.DMA,
        ],
        compiler_params=pltpu.CompilerParams(use_tc_tiling_on_sc=False,
                                             needs_layout_passes=False),
    )
    return run(srcp, dstp, ssrc, sdst, t0, t1, t2, t3)


def _norm_body(m0, m1, m2, m3, ss, sc_ref, t0, t1, t2, t3, b_ref, out_ref):
    bias = b_ref[...]
    ssv = ss[...].astype(jnp.float32)
    wself = sc_ref[:, 4:8]
    parts = []
    for h, (m, t) in enumerate(((m0, t0), (m1, t1), (m2, t2), (m3, t3))):
        mv = m[...].astype(jnp.float32)
        hv = t[...].astype(jnp.float32)
        ws = wself[:, h:h + 1]
        num = mv[0] + mv[1] + ws * hv
        den = ssv[0, :, 2 * h] + ssv[1, :, 2 * h] + wself[:, h]
        g = num / den[:, None] + bias[0, DH * h:DH * h + DH][None, :]
        r = jnp.maximum(g, 0.0)
        parts.append(jnp.sum(r, axis=0))
    out_ref[...] = jnp.concatenate(parts)[None, None, :]


def _normalize_pool(m0, m1, m2, m3, ss, ssrc, t0, t1, t2, t3, bias):
    grid = NF // RB3
    mspec = pl.BlockSpec((2, RB3, DH), lambda i: (0, i, 0))
    return pl.pallas_call(
        _norm_body,
        grid=(grid,),
        in_specs=[mspec] * 5
        + [pl.BlockSpec((RB3, 16), lambda i: (i, 0))]
        + [pl.BlockSpec((RB3, DH), lambda i: (i, 0))] * 4
        + [pl.BlockSpec((1, H), lambda i: (0, 0))],
        out_specs=pl.BlockSpec((1, 1, H), lambda i: (i, 0, 0)),
        out_shape=jax.ShapeDtypeStruct((grid, 1, H), jnp.float32),
    )(m0, m1, m2, m3, ss, ssrc, t0, t1, t2, t3, bias)


def _lstm_body(p_ref, wih_ref, whh_ref, bs_ref, g_ref, bt_ref, out_ref):
    p = p_ref[...]
    pooled = p.reshape(16, N // RB3, H).sum(axis=1) * (1.0 / N)
    seq = pooled.reshape(B, T, H)
    h = jnp.zeros((B, H), jnp.float32)
    c = jnp.zeros((B, H), jnp.float32)
    wih = wih_ref[...]
    whh = whh_ref[...]
    bs = bs_ref[...]
    for t in range(T):
        xt = seq[:, t, :]
        g = (jnp.dot(xt, wih, preferred_element_type=jnp.float32)
             + jnp.dot(h, whh, preferred_element_type=jnp.float32) + bs)
        i = jax.nn.sigmoid(g[:, 0:H])
        f = jax.nn.sigmoid(g[:, H:2 * H])
        gg = jnp.tanh(g[:, 2 * H:3 * H])
        o = jax.nn.sigmoid(g[:, 3 * H:4 * H])
        c = f * c + i * gg
        h = o * jnp.tanh(c)
    mean = jnp.mean(h, axis=-1, keepdims=True)
    var = jnp.mean((h - mean) ** 2, axis=-1, keepdims=True)
    y = (h - mean) / jnp.sqrt(var + 1e-5) * g_ref[...] + bt_ref[...]
    out_ref[...] = jnp.broadcast_to(y[:, None, :], (B, N, H))


def _lstm_ln(partials, wihT, whhT, bsum, gamma, beta):
    return pl.pallas_call(
        _lstm_body,
        out_shape=jax.ShapeDtypeStruct((B, N, H), jnp.float32),
    )(partials, wihT, whhT, bsum, gamma, beta)


def kernel(x, edge_index, W_gat, att_src, att_dst, bias_gat,
           w_ih, w_hh, b_ih, b_hh, ln_gamma, ln_beta):
    x_flat = x.reshape(NF, F)
    src = edge_index[0].astype(jnp.int32)
    dst = edge_index[1].astype(jnp.int32)
    pad = E_PAD - E
    fill = (jnp.arange(pad, dtype=jnp.int32) * 97) % NF
    srcp = jnp.concatenate([src, fill]).reshape(NTILES * NBATCH, BW)
    dstp = jnp.concatenate([dst, fill]).reshape(NTILES * NBATCH, BW)

    eye = jnp.eye(HEADS, dtype=jnp.float32)
    Asrc = (att_src[:, :, None] * eye[:, None, :]).reshape(F, HEADS)
    Adst = (att_dst[:, :, None] * eye[:, None, :]).reshape(F, HEADS)
    Aall = jnp.concatenate([Asrc, Adst], axis=1)

    t0, t1, t2, t3, ssrc, sdst = _project(x_flat, W_gat, Aall)
    m0, m1, m2, m3, ss, _ = _edge_phase(srcp, dstp, ssrc, sdst,
                                        t0, t1, t2, t3)
    m0, m1, m2, m3, ss = (a.reshape(2, NF, DH) for a in (m0, m1, m2, m3, ss))
    partials = _normalize_pool(m0, m1, m2, m3, ss, ssrc, t0, t1, t2, t3,
                               bias_gat.reshape(1, H)).reshape(NF // RB3, H)
    out = _lstm_ln(partials, w_ih.T, w_hh.T,
                   (b_ih + b_hh).reshape(1, 4 * H),
                   ln_gamma.reshape(1, H), ln_beta.reshape(1, H))
    return out
